# baseline (speedup 1.0000x reference)
import functools

import jax
import jax.numpy as jnp
from jax import lax
from jax.experimental import pallas as pl
from jax.experimental.pallas import tpu as pltpu
from jax.experimental.pallas import tpu_sc as plsc

N = 10000
K = 32
D = 128
L = 16
NC = 2
NS = 16
NW = NC * NS
NPW = (N + NW - 1) // NW
DJ = D // L

P_LEN = 272


def _tec_kernel(nb_hbm, self_hbm, par_hbm, out_hbm,
                par_v, nb_v, self_v, pmat_v, out_v):
    wid = lax.axis_index("c") * NS + lax.axis_index("s")
    base = wid * NPW
    count = jnp.minimum(NPW, N - base)

    pltpu.sync_copy(par_hbm, par_v)
    wt = [par_v[pl.ds(j * L, L)] for j in range(DJ)]
    wh = [par_v[pl.ds(D + j * L, L)] for j in range(DJ)]
    tail = par_v[pl.ds(2 * D, L)]
    bt = tail[0]
    bh = tail[1]

    lanes = lax.iota(jnp.int32, L)

    def node_body(i, _):
        n = base + i
        pltpu.sync_copy(nb_hbm.at[n], nb_v)
        pltpu.sync_copy(self_hbm.at[n], self_v)

        def logit_body(k, _):
            acc = nb_v[k, pl.ds(0, L)] * wt[0]
            for j in range(1, DJ):
                acc = acc + nb_v[k, pl.ds(j * L, L)] * wt[j]
            pmat_v[k, :] = acc
            return 0

        lax.fori_loop(0, K, logit_body, 0, unroll=2)

        sacc = self_v[pl.ds(0, L)] * wh[0]
        for j in range(1, DJ):
            sacc = sacc + self_v[pl.ds(j * L, L)] * wh[j]
        c = jnp.sum(sacc) + bh + bt

        att0 = plsc.load_gather(pmat_v, [lanes, jnp.zeros((L,), jnp.int32)])
        att1 = plsc.load_gather(pmat_v, [lanes + L, jnp.zeros((L,), jnp.int32)])
        for l in range(1, L):
            col = jnp.full((L,), l, jnp.int32)
            att0 = att0 + plsc.load_gather(pmat_v, [lanes, col])
            att1 = att1 + plsc.load_gather(pmat_v, [lanes + L, col])
        att0 = att0 + c
        att1 = att1 + c

        att0 = jnp.where(att0 >= 0.0, att0, att0 * 0.2)
        att1 = jnp.where(att1 >= 0.0, att1, att1 * 0.2)
        m = jnp.max(jnp.maximum(att0, att1))
        e0 = jnp.exp(att0 - m)
        e1 = jnp.exp(att1 - m)
        s = jnp.broadcast_to(jnp.sum(e0 + e1), (L,))
        w0 = e0 / s
        w1 = e1 / s

        zeros = jnp.zeros((L,), jnp.float32)
        out = [zeros] * DJ
        for k in range(K):
            wk = (w0 if k < L else w1)[k % L]
            for j in range(DJ):
                out[j] = out[j] + wk * nb_v[k, pl.ds(j * L, L)]
        for j in range(DJ):
            out_v[pl.ds(j * L, L)] = out[j]
        pltpu.sync_copy(out_v, out_hbm.at[n])
        return 0

    lax.fori_loop(0, count, node_body, 0)


@jax.jit
def _sc_agg(neighbor_vectors, self_vector, params):
    mesh = plsc.VectorSubcoreMesh(core_axis_name="c", subcore_axis_name="s")
    f = pl.kernel(
        _tec_kernel,
        mesh=mesh,
        compiler_params=pltpu.CompilerParams(needs_layout_passes=False),
        out_type=jax.ShapeDtypeStruct((N, D), jnp.float32),
        scratch_types=[
            pltpu.VMEM((P_LEN,), jnp.float32),
            pltpu.VMEM((K, D), jnp.float32),
            pltpu.VMEM((D,), jnp.float32),
            pltpu.VMEM((K, L), jnp.float32),
            pltpu.VMEM((D,), jnp.float32),
        ],
    )
    return f(neighbor_vectors, self_vector, params)


def kernel(neighbor_vectors, self_vector, Wh, bh, Wt, bt):
    params = jnp.concatenate([
        Wt.reshape(-1), Wh.reshape(-1), bt.reshape(-1), bh.reshape(-1),
        jnp.zeros((P_LEN - 2 * D - 2,), jnp.float32),
    ])
    return _sc_agg(neighbor_vectors, self_vector, params)


# speedup vs baseline: 2.2167x; 2.2167x over previous
import functools

import jax
import jax.numpy as jnp
from jax import lax
from jax.experimental import pallas as pl
from jax.experimental.pallas import tpu as pltpu
from jax.experimental.pallas import tpu_sc as plsc

N = 10000
K = 32
D = 128
L = 16
NC = 2
NS = 16
NW = NC * NS
NPW = (N + NW - 1) // NW
DJ = D // L
B = 8
NBUF = 2

P_LEN = 272


def _tec_kernel(nb_hbm, self_hbm, par_hbm, out_hbm,
                par_v, nb_buf, self_buf, pmat_v, out_v,
                nbsem0, nbsem1, ssem0, ssem1):
    nbsem = (nbsem0, nbsem1)
    ssem = (ssem0, ssem1)
    wid = lax.axis_index("c") * NS + lax.axis_index("s")
    base = wid * NPW
    count = jnp.minimum(NPW, N - base)
    nblk = (count + B - 1) >> 3
    last_s = base + count - B

    pltpu.sync_copy(par_hbm, par_v)
    wt = [par_v[pl.ds(j * L, L)] for j in range(DJ)]
    wh = [par_v[pl.ds(D + j * L, L)] for j in range(DJ)]
    tail = par_v[pl.ds(2 * D, L)]
    bt = tail[0]
    bh = tail[1]
    lanes = lax.iota(jnp.int32, L)

    def start_of(blk):
        return jnp.minimum(base + blk * B, last_s)

    def fetch(blk, b):
        s = start_of(blk)
        pltpu.async_copy(nb_hbm.at[pl.ds(s, B)], nb_buf.at[b], nbsem[b])
        pltpu.async_copy(self_hbm.at[pl.ds(s, B)], self_buf.at[b], ssem[b])

    def wait_fetch(blk, b):
        s = start_of(blk)
        pltpu.make_async_copy(nb_hbm.at[pl.ds(s, B)], nb_buf.at[b],
                              nbsem[b]).wait()
        pltpu.make_async_copy(self_hbm.at[pl.ds(s, B)], self_buf.at[b],
                              ssem[b]).wait()

    fetch(0, 0)
    fetch(1, 1)

    @pl.loop(0, nblk, step=NBUF)
    def _blocks(g):
        for b in range(NBUF):
            blk = g + b

            @pl.when(blk < nblk)
            def _():
                s = start_of(blk)
                wait_fetch(blk, b)

                @pl.loop(0, B)
                def _node(i):
                    for k in range(K):
                        acc = nb_buf[b, i, k, pl.ds(0, L)] * wt[0]
                        for j in range(1, DJ):
                            acc = acc + nb_buf[b, i, k, pl.ds(j * L, L)] * wt[j]
                        pmat_v[k, :] = acc

                    sacc = self_buf[b, i, 0, pl.ds(0, L)] * wh[0]
                    for j in range(1, DJ):
                        sacc = sacc + self_buf[b, i, 0, pl.ds(j * L, L)] * wh[j]
                    c = jnp.sum(sacc) + bh + bt

                    zcol = jnp.zeros((L,), jnp.int32)
                    att0 = plsc.load_gather(pmat_v, [lanes, zcol])
                    att1 = plsc.load_gather(pmat_v, [lanes + L, zcol])
                    for l in range(1, L):
                        col = jnp.full((L,), l, jnp.int32)
                        att0 = att0 + plsc.load_gather(pmat_v, [lanes, col])
                        att1 = att1 + plsc.load_gather(pmat_v, [lanes + L, col])
                    att0 = att0 + c
                    att1 = att1 + c

                    att0 = jnp.where(att0 >= 0.0, att0, att0 * 0.2)
                    att1 = jnp.where(att1 >= 0.0, att1, att1 * 0.2)
                    m = jnp.max(jnp.maximum(att0, att1))
                    e0 = jnp.exp(att0 - m)
                    e1 = jnp.exp(att1 - m)
                    ssum = jnp.broadcast_to(jnp.sum(e0 + e1), (L,))
                    w0 = e0 / ssum
                    w1 = e1 / ssum

                    zeros = jnp.zeros((L,), jnp.float32)
                    out = [zeros] * DJ
                    for k in range(K):
                        wk = (w0 if k < L else w1)[k % L]
                        for j in range(DJ):
                            out[j] = out[j] + wk * nb_buf[b, i, k,
                                                          pl.ds(j * L, L)]
                    for j in range(DJ):
                        out_v[i, 0, pl.ds(j * L, L)] = out[j]

                pltpu.sync_copy(out_v, out_hbm.at[pl.ds(s, B)])

                @pl.when(blk + NBUF < nblk)
                def _():
                    fetch(blk + NBUF, b)


@jax.jit
def _sc_agg(neighbor_vectors, self_vector, params):
    mesh = plsc.VectorSubcoreMesh(core_axis_name="c", subcore_axis_name="s")
    f = pl.kernel(
        _tec_kernel,
        mesh=mesh,
        compiler_params=pltpu.CompilerParams(needs_layout_passes=False),
        out_type=jax.ShapeDtypeStruct((N, 1, D), jnp.float32),
        scratch_types=[
            pltpu.VMEM((P_LEN,), jnp.float32),
            pltpu.VMEM((NBUF, B, K, D), jnp.float32),
            pltpu.VMEM((NBUF, B, 1, D), jnp.float32),
            pltpu.VMEM((K, L), jnp.float32),
            pltpu.VMEM((B, 1, D), jnp.float32),
            pltpu.SemaphoreType.DMA,
            pltpu.SemaphoreType.DMA,
            pltpu.SemaphoreType.DMA,
            pltpu.SemaphoreType.DMA,
        ],
    )
    return f(neighbor_vectors, self_vector.reshape(N, 1, D),
             params).reshape(N, D)


def kernel(neighbor_vectors, self_vector, Wh, bh, Wt, bt):
    params = jnp.concatenate([
        Wt.reshape(-1), Wh.reshape(-1), bt.reshape(-1), bh.reshape(-1),
        jnp.zeros((P_LEN - 2 * D - 2,), jnp.float32),
    ])
    return _sc_agg(neighbor_vectors, self_vector, params)
